# Initial kernel scaffold; baseline (speedup 1.0000x reference)
import jax
import jax.numpy as jnp
from jax.experimental import pallas as pl


def kernel(conduit_area, melt_rate, cell_area, bedrock_elevation, water_pressure, surface_slope, bedrock_slope, status_at_node, edge_index):
    raise NotImplementedError("write your pallas kernel here")


# speedup vs baseline: 131.0325x; 131.0325x over previous
import dataclasses
import functools

import jax
import jax.numpy as jnp
from jax import lax
from jax.experimental import pallas as pl
from jax.experimental.pallas import tpu as pltpu
from jax.experimental.pallas import tpu_sc as plsc

SEC_PER_A = 31556926.0
RHO_W = 1000.0
RHO_I = 917.0
GRAVITY = 9.81
FLOW_COEFF = 0.0405
FLOW_EXP = 1.25
N_FLOW_ITERS = 10

NN = 100000
NPAD = 100352
EE = 1600000
NWORK = 32
ROWS_PER_TILE = 400
EROWS = NWORK * ROWS_PER_TILE
EPAD = EROWS * 128
CH_ROWS = 16
N_CHUNKS = ROWS_PER_TILE // CH_ROWS

_SC_MESH = plsc.VectorSubcoreMesh(core_axis_name="c", subcore_axis_name="s")

_SC_PARAMS = pltpu.CompilerParams()
if "needs_layout_passes" in pltpu.CompilerParams.__dataclass_fields__:
    _SC_PARAMS = dataclasses.replace(_SC_PARAMS, needs_layout_passes=False)


def _f32(shape):
    return jax.ShapeDtypeStruct(shape, jnp.float32)


def _tc_elementwise(body, n_out, *arrays):
    rows = NPAD // 128
    outs = pl.pallas_call(
        body,
        out_shape=[jax.ShapeDtypeStruct((rows, 128), jnp.float32)] * n_out,
    )(*[a.reshape(rows, 128) for a in arrays])
    return [o.reshape(NPAD) for o in outs]


def _pre_body(mr_ref, ca_ref, be_ref, wp_ref, ss_ref, bs_ref,
              melt_ref, pot_ref, geo_ref):
    melt_ref[...] = mr_ref[...] * ca_ref[...] * (1.0 / SEC_PER_A)
    pot_ref[...] = RHO_W * GRAVITY * be_ref[...] + wp_ref[...]
    geo_ref[...] = (-RHO_I * GRAVITY * ss_ref[...]
                    - (RHO_W - RHO_I) * GRAVITY * bs_ref[...])


def _gdis_body(melt_ref, i0_ref, i1_ref, ws0_ref, ws1_ref, g_ref, dis_ref):
    dis = melt_ref[...] + (i0_ref[...] + i1_ref[...])
    dis_ref[...] = dis
    g_ref[...] = dis / (ws0_ref[...] + ws1_ref[...] + 1e-12)


def _fin_body(dis_ref, ca_ref, st_ref, hg_ref):
    grad = (dis_ref[...] * FLOW_COEFF * ca_ref[...] ** FLOW_EXP) ** 2
    hg_ref[...] = jnp.where(st_ref[...] == 0.0, grad, 0.0)


def _core_writeback(cid, sid, shared, out0_hbm, out1_hbm):
    @pl.when(sid == 0)
    def _():
        @pl.when(cid == 0)
        def _():
            pltpu.sync_copy(shared, out0_hbm)

        @pl.when(cid == 1)
        def _():
            pltpu.sync_copy(shared, out1_hbm)


def _scatter_add_chunk(obuf, idxbuf, shared, sem):
    descs = [pltpu.async_copy(obuf.at[j], shared.at[idxbuf.at[j]], sem,
                              add=True)
             for j in range(CH_ROWS)]
    for d in descs:
        d.wait()


def _wsum_kernel(pot_hbm, src_hbm, dst_hbm, zer_hbm,
                 w_hbm, ws0_hbm, ws1_hbm,
                 node_v, ibuf, dbuf, obuf, shared, sem, wsem):
    cid = lax.axis_index("c")
    sid = lax.axis_index("s")
    wid = sid * 2 + cid

    @pl.when(sid == 0)
    def _():
        pltpu.sync_copy(zer_hbm, shared)
    pltpu.async_copy(pot_hbm, node_v, wsem).wait()
    plsc.subcore_barrier()

    base_row = wid * ROWS_PER_TILE

    @pl.loop(0, N_CHUNKS)
    def _(ci):
        r0 = base_row + ci * CH_ROWS
        pltpu.sync_copy(src_hbm.at[pl.ds(r0, CH_ROWS)], ibuf)
        pltpu.sync_copy(dst_hbm.at[pl.ds(r0, CH_ROWS)], dbuf)

        for j in range(CH_ROWS):
            @pl.loop(0, 128, step=16)
            def _(k):
                sl = pl.ds(k, 16)
                ps = plsc.load_gather(node_v, [ibuf[j, sl]])
                pd = plsc.load_gather(node_v, [dbuf[j, sl]])
                obuf[j, sl] = jnp.maximum(ps - pd, 0.0)

        pltpu.sync_copy(obuf, w_hbm.at[pl.ds(r0, CH_ROWS)])
        _scatter_add_chunk(obuf, ibuf, shared, sem)

    plsc.subcore_barrier()
    _core_writeback(cid, sid, shared, ws0_hbm, ws1_hbm)


def _iter_kernel(g_hbm, src_hbm, w_hbm, dst_hbm, zer_hbm,
                 i0_hbm, i1_hbm,
                 node_v, ibuf, dbuf, wbuf, shared, sem, wsem):
    cid = lax.axis_index("c")
    sid = lax.axis_index("s")
    wid = sid * 2 + cid

    @pl.when(sid == 0)
    def _():
        pltpu.sync_copy(zer_hbm, shared)
    pltpu.async_copy(g_hbm, node_v, wsem).wait()
    plsc.subcore_barrier()

    base_row = wid * ROWS_PER_TILE

    @pl.loop(0, N_CHUNKS)
    def _(ci):
        r0 = base_row + ci * CH_ROWS
        pltpu.sync_copy(src_hbm.at[pl.ds(r0, CH_ROWS)], ibuf)
        pltpu.sync_copy(w_hbm.at[pl.ds(r0, CH_ROWS)], wbuf)
        pltpu.sync_copy(dst_hbm.at[pl.ds(r0, CH_ROWS)], dbuf)

        for j in range(CH_ROWS):
            @pl.loop(0, 128, step=16)
            def _(k):
                sl = pl.ds(k, 16)
                gv = plsc.load_gather(node_v, [ibuf[j, sl]])
                wbuf[j, sl] = gv * wbuf[j, sl]

        _scatter_add_chunk(wbuf, dbuf, shared, sem)

    plsc.subcore_barrier()
    _core_writeback(cid, sid, shared, i0_hbm, i1_hbm)


def kernel(conduit_area, melt_rate, cell_area, bedrock_elevation,
           water_pressure, surface_slope, bedrock_slope,
           status_at_node, edge_index):
    f32 = jnp.float32
    npad = NPAD - NN
    ca = jnp.pad(conduit_area, (0, npad))
    mr = jnp.pad(melt_rate, (0, npad))
    cla = jnp.pad(cell_area, (0, npad))
    be = jnp.pad(bedrock_elevation, (0, npad))
    wp = jnp.pad(water_pressure, (0, npad))
    ss = jnp.pad(surface_slope, (0, npad))
    bs = jnp.pad(bedrock_slope, (0, npad))
    st = jnp.pad(status_at_node, (0, npad))

    epad = EPAD - EE
    pad_ids = (jnp.arange(epad, dtype=jnp.int32) * 61) % NN
    src = jnp.concatenate([edge_index[0], pad_ids]).reshape(EROWS, 128)
    dst = jnp.concatenate([edge_index[1], pad_ids]).reshape(EROWS, 128)
    zer = jnp.zeros((NPAD,), f32)

    melt, pot, geo = _tc_elementwise(_pre_body, 3, mr, cla, be, wp, ss, bs)

    wsum_fn = pl.kernel(
        _wsum_kernel,
        out_type=[_f32((EROWS, 128)), _f32((NPAD,)), _f32((NPAD,))],
        mesh=_SC_MESH,
        compiler_params=_SC_PARAMS,
        scratch_types=[
            pltpu.VMEM((NPAD,), jnp.float32),
            pltpu.VMEM((CH_ROWS, 128), jnp.int32),
            pltpu.VMEM((CH_ROWS, 128), jnp.int32),
            pltpu.VMEM((CH_ROWS, 128), jnp.float32),
            pltpu.VMEM_SHARED((NPAD,), jnp.float32),
            pltpu.SemaphoreType.DMA,
            pltpu.SemaphoreType.DMA,
        ],
    )
    w, ws0, ws1 = wsum_fn(pot, src, dst, zer)

    iter_fn = pl.kernel(
        _iter_kernel,
        out_type=[_f32((NPAD,)), _f32((NPAD,))],
        mesh=_SC_MESH,
        compiler_params=_SC_PARAMS,
        scratch_types=[
            pltpu.VMEM((NPAD,), jnp.float32),
            pltpu.VMEM((CH_ROWS, 128), jnp.int32),
            pltpu.VMEM((CH_ROWS, 128), jnp.int32),
            pltpu.VMEM((CH_ROWS, 128), jnp.float32),
            pltpu.VMEM_SHARED((NPAD,), jnp.float32),
            pltpu.SemaphoreType.DMA,
            pltpu.SemaphoreType.DMA,
        ],
    )

    gdis_fn = functools.partial(_tc_elementwise, _gdis_body, 2)
    inf0 = zer
    inf1 = zer
    dis = melt
    for _ in range(N_FLOW_ITERS):
        g, dis = gdis_fn(melt, inf0, inf1, ws0, ws1)
        inf0, inf1 = iter_fn(g, src, w, dst, zer)
    _, dis = gdis_fn(melt, inf0, inf1, ws0, ws1)

    hg = _tc_elementwise(_fin_body, 1, dis, ca, st.astype(f32))[0]

    return (hg[:NN], dis[:NN], pot[:NN], geo[:NN])
